# Initial kernel scaffold; baseline (speedup 1.0000x reference)
import jax
import jax.numpy as jnp
from jax.experimental import pallas as pl


def kernel(x, edge_index, edge_attr, batch, W0, b0, Wn1, bn1, Wn2, bn2, root, conv_bias, Wih, Whh, bih, bhh, Wih2, Whh2, bih2, bhh2, W1, b1, W2, b2):
    raise NotImplementedError("write your pallas kernel here")


# speedup vs baseline: 1.9325x; 1.9325x over previous
import functools

import jax
import jax.numpy as jnp
from jax import lax
from jax.experimental import pallas as pl
from jax.experimental.pallas import tpu as pltpu
from jax.experimental.pallas import tpu_sc as plsc

_N = 10000
_E = 160000
_NODE_F = 128
_EDGE_F = 16
_DIM = 32
_NN_HID = 128
_NG = 64
_ITERS = 3

_NW = 32
_CHUNK = 128
_EPAD = 163840
_CPW = _EPAD // _NW // _CHUNK
_NPAD = 10240
_ROWS_PT = _NPAD // 16


def _sc_gather(table, idx2):
    mesh = plsc.VectorSubcoreMesh(core_axis_name="c", subcore_axis_name="s")

    @functools.partial(
        pl.kernel, mesh=mesh,
        out_type=jax.ShapeDtypeStruct((_EPAD, 128), jnp.float32),
        scratch_types=[
            pltpu.VMEM((_CPW, _CHUNK), jnp.int32),
            pltpu.VMEM((_CHUNK, 128), jnp.float32),
            pltpu.SemaphoreType.DMA,
        ],
    )
    def k(table_hbm, idx_hbm, out_hbm, idx_v, rows_v, sem):
        wid = lax.axis_index("s") * 2 + lax.axis_index("c")
        pltpu.sync_copy(idx_hbm.at[wid], idx_v)
        base = wid * (_CPW * _CHUNK)

        def body(ci, carry):
            pltpu.async_copy(table_hbm.at[idx_v.at[ci]], rows_v, sem).wait()
            pltpu.sync_copy(rows_v, out_hbm.at[pl.ds(base + ci * _CHUNK, _CHUNK)])
            return carry

        lax.fori_loop(0, _CPW, body, 0)

    return k(table, idx2)


def _sc_scatter_add(vals, idx2, zeros):
    mesh = plsc.VectorSubcoreMesh(core_axis_name="c", subcore_axis_name="s")

    @functools.partial(
        pl.kernel, mesh=mesh,
        out_type=jax.ShapeDtypeStruct((2 * _NPAD, _DIM), jnp.float32),
        scratch_types=[
            pltpu.VMEM((_CPW, _CHUNK), jnp.int32),
            pltpu.VMEM((_CHUNK, _DIM), jnp.float32),
            pltpu.VMEM_SHARED((_NPAD, _DIM), jnp.float32),
        ],
    )
    def k(vals_hbm, idx_hbm, zeros_hbm, out_hbm, idx_v, vals_v, table):
        c = lax.axis_index("c")
        s = lax.axis_index("s")
        wid = s * 2 + c
        r0 = s * _ROWS_PT
        pltpu.sync_copy(zeros_hbm.at[pl.ds(r0, _ROWS_PT)],
                        table.at[pl.ds(r0, _ROWS_PT)])
        pltpu.sync_copy(idx_hbm.at[wid], idx_v)
        plsc.subcore_barrier()
        base = wid * (_CPW * _CHUNK)

        def body(ci, carry):
            pltpu.sync_copy(vals_hbm.at[pl.ds(base + ci * _CHUNK, _CHUNK)], vals_v)
            pltpu.sync_copy(vals_v, table.at[idx_v.at[ci]], add=True)
            return carry

        lax.fori_loop(0, _CPW, body, 0)
        plsc.subcore_barrier()
        pltpu.sync_copy(table.at[pl.ds(r0, _ROWS_PT)],
                        out_hbm.at[pl.ds(c * _NPAD + r0, _ROWS_PT)])

    return k(vals, idx2, zeros)


def _mm_bias_relu(x, wt, br, blk):
    rows, kdim = x.shape
    cols = wt.shape[1]

    def body(x_ref, w_ref, b_ref, o_ref):
        o_ref[...] = jax.nn.relu(
            jnp.dot(x_ref[...], w_ref[...], preferred_element_type=jnp.float32)
            + b_ref[...])

    return pl.pallas_call(
        body,
        grid=(rows // blk,),
        in_specs=[
            pl.BlockSpec((blk, kdim), lambda i: (i, 0)),
            pl.BlockSpec((kdim, cols), lambda i: (0, 0)),
            pl.BlockSpec((1, cols), lambda i: (0, 0)),
        ],
        out_specs=pl.BlockSpec((blk, cols), lambda i: (i, 0)),
        out_shape=jax.ShapeDtypeStruct((rows, cols), jnp.float32),
    )(x, wt, br)


def _msg_kernel(hid, xj, wn2t, bn2r, rep, summ):
    blk = 640

    def body(h_ref, x_ref, w_ref, b_ref, r_ref, s_ref, o_ref):
        we = jnp.dot(h_ref[...], w_ref[...], preferred_element_type=jnp.float32)
        we = we + b_ref[...]
        xr = jnp.dot(x_ref[...], r_ref[...], preferred_element_type=jnp.float32)
        o_ref[...] = jnp.dot(we * xr, s_ref[...],
                             preferred_element_type=jnp.float32)

    return pl.pallas_call(
        body,
        grid=(_EPAD // blk,),
        in_specs=[
            pl.BlockSpec((blk, _NN_HID), lambda i: (i, 0)),
            pl.BlockSpec((blk, 128), lambda i: (i, 0)),
            pl.BlockSpec((_NN_HID, _DIM * _DIM), lambda i: (0, 0)),
            pl.BlockSpec((1, _DIM * _DIM), lambda i: (0, 0)),
            pl.BlockSpec((128, _DIM * _DIM), lambda i: (0, 0)),
            pl.BlockSpec((_DIM * _DIM, _DIM), lambda i: (0, 0)),
        ],
        out_specs=pl.BlockSpec((blk, _DIM), lambda i: (i, 0)),
        out_shape=jax.ShapeDtypeStruct((_EPAD, _DIM), jnp.float32),
    )(hid, xj, wn2t, bn2r, rep, summ)


def _node_update(aggp, degp, s, root, cbr, wiht, whht, bihr, bhhr):
    blk = 1000

    def body(a0, a1, d0, d1, s_ref, rt, cb, wi, wh, bi, bh, o_ref):
        deg = d0[0] + d1[0]
        denom = jnp.maximum(deg, 1.0)
        agg = (a0[0] + a1[0]) / denom
        sv = s_ref[...][:, :_DIM]
        m = jax.nn.relu(
            jnp.dot(sv, rt[...], preferred_element_type=jnp.float32) + agg
            + cb[...])
        gi = jnp.dot(m, wi[...], preferred_element_type=jnp.float32) + bi[...]
        gh = jnp.dot(sv, wh[...], preferred_element_type=jnp.float32) + bh[...]
        r = jax.nn.sigmoid(gi[:, :_DIM] + gh[:, :_DIM])
        z = jax.nn.sigmoid(gi[:, _DIM:2 * _DIM] + gh[:, _DIM:2 * _DIM])
        n = jnp.tanh(gi[:, 2 * _DIM:] + r * gh[:, 2 * _DIM:])
        h_new = (1.0 - z) * n + z * sv
        o_ref[...] = jnp.concatenate(
            [h_new, jnp.zeros((h_new.shape[0], 128 - _DIM), jnp.float32)],
            axis=1)

    part = pl.BlockSpec((1, blk, _DIM), lambda i: (0, i, 0))
    part1 = pl.BlockSpec((1, blk, _DIM), lambda i: (1, i, 0))
    return pl.pallas_call(
        body,
        grid=(_N // blk,),
        in_specs=[
            part, part1, part, part1,
            pl.BlockSpec((blk, 128), lambda i: (i, 0)),
            pl.BlockSpec((_DIM, _DIM), lambda i: (0, 0)),
            pl.BlockSpec((1, _DIM), lambda i: (0, 0)),
            pl.BlockSpec((_DIM, 3 * _DIM), lambda i: (0, 0)),
            pl.BlockSpec((_DIM, 3 * _DIM), lambda i: (0, 0)),
            pl.BlockSpec((1, 3 * _DIM), lambda i: (0, 0)),
            pl.BlockSpec((1, 3 * _DIM), lambda i: (0, 0)),
        ],
        out_specs=pl.BlockSpec((blk, 128), lambda i: (i, 0)),
        out_shape=jax.ShapeDtypeStruct((_N, 128), jnp.float32),
    )(aggp, aggp, degp, degp, s, root, cbr, wiht, whht, bihr, bhhr)


def _set2set(s, batch2, wih2t, whh2t, bgr, w1t, b1r, w2t, b2r):
    def body(s_ref, b_ref, wi, wh, bg, w1, b1, w2, b2, o_ref):
        sv = s_ref[...][:, :_DIM]
        bcol = b_ref[...]
        gidx = lax.broadcasted_iota(jnp.int32, (1, _NG), 1)
        oneb = (bcol == gidx)
        onef = oneb.astype(jnp.float32)
        q_star = jnp.zeros((_NG, 2 * _DIM), jnp.float32)
        hs = jnp.zeros((_NG, _DIM), jnp.float32)
        cs = jnp.zeros((_NG, _DIM), jnp.float32)
        for _ in range(_ITERS):
            gates = (jnp.dot(q_star, wi[...], preferred_element_type=jnp.float32)
                     + jnp.dot(hs, wh[...], preferred_element_type=jnp.float32)
                     + bg[...])
            i_g = jax.nn.sigmoid(gates[:, :_DIM])
            f_g = jax.nn.sigmoid(gates[:, _DIM:2 * _DIM])
            g_g = jnp.tanh(gates[:, 2 * _DIM:3 * _DIM])
            o_g = jax.nn.sigmoid(gates[:, 3 * _DIM:])
            cs = f_g * cs + i_g * g_g
            hs = o_g * jnp.tanh(cs)
            q = hs
            qb = jnp.dot(onef, q, preferred_element_type=jnp.float32)
            e = jnp.sum(sv * qb, axis=1, keepdims=True)
            masked = jnp.where(oneb, e, -1e30)
            emax = jnp.max(masked, axis=0, keepdims=True)
            emax_b = lax.dot_general(onef, emax, (((1,), (1,)), ((), ())),
                                     preferred_element_type=jnp.float32)
            ee = jnp.exp(e - emax_b)
            esum = lax.dot_general(ee, onef, (((0,), (0,)), ((), ())),
                                   preferred_element_type=jnp.float32)
            esum_b = lax.dot_general(onef, esum, (((1,), (1,)), ((), ())),
                                     preferred_element_type=jnp.float32)
            a = ee / esum_b
            rsum = lax.dot_general(onef, a * sv, (((0,), (0,)), ((), ())),
                                   preferred_element_type=jnp.float32)
            q_star = jnp.concatenate([q, rsum], axis=1)
        o1 = jax.nn.relu(
            jnp.dot(q_star, w1[...], preferred_element_type=jnp.float32)
            + b1[...])
        o_ref[...] = (jnp.dot(o1, w2[...], preferred_element_type=jnp.float32)
                      + b2[...])

    return pl.pallas_call(
        body,
        out_shape=jax.ShapeDtypeStruct((_NG, 128), jnp.float32),
    )(s, batch2, wih2t, whh2t, bgr, w1t, b1r, w2t, b2r)


def kernel(x, edge_index, edge_attr, batch, W0, b0, Wn1, bn1, Wn2, bn2, root,
           conv_bias, Wih, Whh, bih, bhh, Wih2, Whh2, bih2, bhh2, W1, b1, W2,
           b2):
    src = jnp.concatenate(
        [edge_index[0], jnp.zeros((_EPAD - _E,), jnp.int32)]).reshape(
            _NW, _CPW, _CHUNK)
    dst = jnp.concatenate(
        [edge_index[1], jnp.full((_EPAD - _E,), _NPAD - 1, jnp.int32)]).reshape(
            _NW, _CPW, _CHUNK)
    eap = jnp.concatenate(
        [edge_attr, jnp.zeros((_EPAD - _E, _EDGE_F), jnp.float32)])
    jj = jnp.arange(_DIM * _DIM, dtype=jnp.int32)
    rep = (jj[None, :] // _DIM
           == jnp.arange(128, dtype=jnp.int32)[:, None]).astype(jnp.float32)
    summ = (jj[:, None] % _DIM
            == jnp.arange(_DIM, dtype=jnp.int32)[None, :]).astype(jnp.float32)
    zeros_tab = jnp.zeros((_NPAD, _DIM), jnp.float32)
    ones_vals = jnp.ones((_EPAD, _DIM), jnp.float32)
    w2t_pad = jnp.concatenate(
        [W2.T, jnp.zeros((_DIM, 128 - W2.shape[0]), jnp.float32)], axis=1)
    b2_pad = jnp.concatenate(
        [b2, jnp.zeros((128 - b2.shape[0],), jnp.float32)])[None, :]
    w0t_pad = jnp.concatenate(
        [W0.T, jnp.zeros((_NODE_F, 128 - _DIM), jnp.float32)], axis=1)
    b0_pad = jnp.concatenate(
        [b0, jnp.zeros((128 - _DIM,), jnp.float32)])[None, :]

    s = _mm_bias_relu(x, w0t_pad, b0_pad, 1000)
    hid = _mm_bias_relu(eap, Wn1.T, bn1[None, :], 2048)

    degp = _sc_scatter_add(ones_vals, dst, zeros_tab).reshape(2, _NPAD, _DIM)

    for _ in range(_ITERS):
        xj = _sc_gather(s, src)
        msg = _msg_kernel(hid, xj, Wn2.T, bn2[None, :], rep, summ)
        aggp = _sc_scatter_add(msg, dst, zeros_tab).reshape(2, _NPAD, _DIM)
        s = _node_update(aggp, degp, s, root, conv_bias[None, :],
                         Wih.T, Whh.T, bih[None, :], bhh[None, :])

    out = _set2set(s, batch[:, None], Wih2.T, Whh2.T,
                   (bih2 + bhh2)[None, :], W1.T, b1[None, :], w2t_pad, b2_pad)
    return out[:, :1]


# speedup vs baseline: 1.9714x; 1.0201x over previous
import functools

import jax
import jax.numpy as jnp
from jax import lax
from jax.experimental import pallas as pl
from jax.experimental.pallas import tpu as pltpu
from jax.experimental.pallas import tpu_sc as plsc

_N = 10000
_E = 160000
_NODE_F = 128
_EDGE_F = 16
_DIM = 32
_NN_HID = 128
_NG = 64
_ITERS = 3

_NW = 32
_CHUNK = 128
_EPAD = 163840
_CPW = _EPAD // _NW // _CHUNK
_NPAD = 10240
_ROWS_PT = _NPAD // 16
_GRP = 2
_NGRP = _CPW // _GRP


def _sc_gather(table, idx2):
    mesh = plsc.VectorSubcoreMesh(core_axis_name="c", subcore_axis_name="s")

    @functools.partial(
        pl.kernel, mesh=mesh,
        out_type=jax.ShapeDtypeStruct((_EPAD, 128), jnp.float32),
        scratch_types=[
            pltpu.VMEM((_CPW, _CHUNK), jnp.int32),
            pltpu.VMEM((_CHUNK, 128), jnp.float32),
            pltpu.VMEM((_CHUNK, 128), jnp.float32),
            pltpu.SemaphoreType.DMA,
            pltpu.SemaphoreType.DMA,
            pltpu.SemaphoreType.DMA,
        ],
    )
    def k(table_hbm, idx_hbm, out_hbm, idx_v, rows_v0, rows_v1,
          sem_g0, sem_g1, sem_w):
        wid = lax.axis_index("s") * 2 + lax.axis_index("c")
        pltpu.sync_copy(idx_hbm.at[wid], idx_v)
        base = wid * (_CPW * _CHUNK)

        def body(ci, carry):
            pltpu.async_copy(table_hbm.at[idx_v.at[ci]], rows_v0, sem_g0).wait()
            pltpu.sync_copy(rows_v0,
                            out_hbm.at[pl.ds(base + ci * _CHUNK, _CHUNK)])
            return carry

        lax.fori_loop(0, _CPW, body, 0)

    return k(table, idx2)


def _sc_scatter_add(vals, idx2, zeros):
    mesh = plsc.VectorSubcoreMesh(core_axis_name="c", subcore_axis_name="s")

    @functools.partial(
        pl.kernel, mesh=mesh,
        out_type=jax.ShapeDtypeStruct((2 * _NPAD, _DIM), jnp.float32),
        scratch_types=[
            pltpu.VMEM((_CPW, _CHUNK), jnp.int32),
            pltpu.VMEM((_CHUNK, _DIM), jnp.float32),
            pltpu.VMEM((_CHUNK, _DIM), jnp.float32),
            pltpu.VMEM_SHARED((_NPAD, _DIM), jnp.float32),
            pltpu.SemaphoreType.DMA,
            pltpu.SemaphoreType.DMA,
            pltpu.SemaphoreType.DMA,
        ],
    )
    def k(vals_hbm, idx_hbm, zeros_hbm, out_hbm, idx_v, vals_v0, vals_v1,
          table, sem_r0, sem_r1, sem_a):
        c = lax.axis_index("c")
        s = lax.axis_index("s")
        wid = s * 2 + c
        r0 = s * _ROWS_PT
        pltpu.sync_copy(zeros_hbm.at[pl.ds(r0, _ROWS_PT)],
                        table.at[pl.ds(r0, _ROWS_PT)])
        pltpu.sync_copy(idx_hbm.at[wid], idx_v)
        plsc.subcore_barrier()
        base = wid * (_CPW * _CHUNK)
        sems = (sem_r0, sem_r1)
        bufs = (vals_v0, vals_v1)
        pend = pltpu.async_copy(vals_hbm.at[pl.ds(base, _CHUNK)], bufs[0],
                                sems[0])
        for g in range(_CPW):
            if g + 1 < _CPW:
                nxt = pltpu.async_copy(
                    vals_hbm.at[pl.ds(base + (g + 1) * _CHUNK, _CHUNK)],
                    bufs[(g + 1) % 2], sems[(g + 1) % 2])
            pend.wait()
            pltpu.sync_copy(bufs[g % 2], table.at[idx_v.at[g]], add=True)
            if g + 1 < _CPW:
                pend = nxt
        plsc.subcore_barrier()
        pltpu.sync_copy(table.at[pl.ds(r0, _ROWS_PT)],
                        out_hbm.at[pl.ds(c * _NPAD + r0, _ROWS_PT)])

    return k(vals, idx2, zeros)


def _mm_bias_relu(x, wt, br, blk):
    rows, kdim = x.shape
    cols = wt.shape[1]

    def body(x_ref, w_ref, b_ref, o_ref):
        o_ref[...] = jax.nn.relu(
            jnp.dot(x_ref[...], w_ref[...], preferred_element_type=jnp.float32)
            + b_ref[...])

    return pl.pallas_call(
        body,
        grid=(rows // blk,),
        in_specs=[
            pl.BlockSpec((blk, kdim), lambda i: (i, 0)),
            pl.BlockSpec((kdim, cols), lambda i: (0, 0)),
            pl.BlockSpec((1, cols), lambda i: (0, 0)),
        ],
        out_specs=pl.BlockSpec((blk, cols), lambda i: (i, 0)),
        out_shape=jax.ShapeDtypeStruct((rows, cols), jnp.float32),
    )(x, wt, br)


def _msg_kernel(hid, xj, wn2t, bn2r, rep, summ):
    blk = 640

    def body(h_ref, x_ref, w_ref, b_ref, r_ref, s_ref, o_ref):
        we = jnp.dot(h_ref[...], w_ref[...], preferred_element_type=jnp.float32)
        we = we + b_ref[...]
        xr = jnp.dot(x_ref[...], r_ref[...], preferred_element_type=jnp.float32)
        o_ref[...] = jnp.dot(we * xr, s_ref[...],
                             preferred_element_type=jnp.float32)

    return pl.pallas_call(
        body,
        grid=(_EPAD // blk,),
        in_specs=[
            pl.BlockSpec((blk, _NN_HID), lambda i: (i, 0)),
            pl.BlockSpec((blk, 128), lambda i: (i, 0)),
            pl.BlockSpec((_NN_HID, _DIM * _DIM), lambda i: (0, 0)),
            pl.BlockSpec((1, _DIM * _DIM), lambda i: (0, 0)),
            pl.BlockSpec((128, _DIM * _DIM), lambda i: (0, 0)),
            pl.BlockSpec((_DIM * _DIM, _DIM), lambda i: (0, 0)),
        ],
        out_specs=pl.BlockSpec((blk, _DIM), lambda i: (i, 0)),
        out_shape=jax.ShapeDtypeStruct((_EPAD, _DIM), jnp.float32),
    )(hid, xj, wn2t, bn2r, rep, summ)


def _node_update(aggp, degp, s, root, cbr, wiht, whht, bihr, bhhr):
    blk = 1000

    def body(a0, a1, d0, d1, s_ref, rt, cb, wi, wh, bi, bh, o_ref):
        deg = d0[0] + d1[0]
        denom = jnp.maximum(deg, 1.0)
        agg = (a0[0] + a1[0]) / denom
        sv = s_ref[...][:, :_DIM]
        m = jax.nn.relu(
            jnp.dot(sv, rt[...], preferred_element_type=jnp.float32) + agg
            + cb[...])
        gi = jnp.dot(m, wi[...], preferred_element_type=jnp.float32) + bi[...]
        gh = jnp.dot(sv, wh[...], preferred_element_type=jnp.float32) + bh[...]
        r = jax.nn.sigmoid(gi[:, :_DIM] + gh[:, :_DIM])
        z = jax.nn.sigmoid(gi[:, _DIM:2 * _DIM] + gh[:, _DIM:2 * _DIM])
        n = jnp.tanh(gi[:, 2 * _DIM:] + r * gh[:, 2 * _DIM:])
        h_new = (1.0 - z) * n + z * sv
        o_ref[...] = jnp.concatenate(
            [h_new, jnp.zeros((h_new.shape[0], 128 - _DIM), jnp.float32)],
            axis=1)

    part = pl.BlockSpec((1, blk, _DIM), lambda i: (0, i, 0))
    part1 = pl.BlockSpec((1, blk, _DIM), lambda i: (1, i, 0))
    return pl.pallas_call(
        body,
        grid=(_N // blk,),
        in_specs=[
            part, part1, part, part1,
            pl.BlockSpec((blk, 128), lambda i: (i, 0)),
            pl.BlockSpec((_DIM, _DIM), lambda i: (0, 0)),
            pl.BlockSpec((1, _DIM), lambda i: (0, 0)),
            pl.BlockSpec((_DIM, 3 * _DIM), lambda i: (0, 0)),
            pl.BlockSpec((_DIM, 3 * _DIM), lambda i: (0, 0)),
            pl.BlockSpec((1, 3 * _DIM), lambda i: (0, 0)),
            pl.BlockSpec((1, 3 * _DIM), lambda i: (0, 0)),
        ],
        out_specs=pl.BlockSpec((blk, 128), lambda i: (i, 0)),
        out_shape=jax.ShapeDtypeStruct((_N, 128), jnp.float32),
    )(aggp, aggp, degp, degp, s, root, cbr, wiht, whht, bihr, bhhr)


def _set2set(s, batch2, wih2t, whh2t, bgr, w1t, b1r, w2t, b2r):
    def body(s_ref, b_ref, wi, wh, bg, w1, b1, w2, b2, o_ref):
        sv = s_ref[...][:, :_DIM]
        bcol = b_ref[...]
        gidx = lax.broadcasted_iota(jnp.int32, (1, _NG), 1)
        oneb = (bcol == gidx)
        onef = oneb.astype(jnp.float32)
        q_star = jnp.zeros((_NG, 2 * _DIM), jnp.float32)
        hs = jnp.zeros((_NG, _DIM), jnp.float32)
        cs = jnp.zeros((_NG, _DIM), jnp.float32)
        for _ in range(_ITERS):
            gates = (jnp.dot(q_star, wi[...], preferred_element_type=jnp.float32)
                     + jnp.dot(hs, wh[...], preferred_element_type=jnp.float32)
                     + bg[...])
            i_g = jax.nn.sigmoid(gates[:, :_DIM])
            f_g = jax.nn.sigmoid(gates[:, _DIM:2 * _DIM])
            g_g = jnp.tanh(gates[:, 2 * _DIM:3 * _DIM])
            o_g = jax.nn.sigmoid(gates[:, 3 * _DIM:])
            cs = f_g * cs + i_g * g_g
            hs = o_g * jnp.tanh(cs)
            q = hs
            qb = jnp.dot(onef, q, preferred_element_type=jnp.float32)
            e = jnp.sum(sv * qb, axis=1, keepdims=True)
            masked = jnp.where(oneb, e, -1e30)
            emax = jnp.max(masked, axis=0, keepdims=True)
            emax_b = lax.dot_general(onef, emax, (((1,), (1,)), ((), ())),
                                     preferred_element_type=jnp.float32)
            ee = jnp.exp(e - emax_b)
            esum = lax.dot_general(ee, onef, (((0,), (0,)), ((), ())),
                                   preferred_element_type=jnp.float32)
            esum_b = lax.dot_general(onef, esum, (((1,), (1,)), ((), ())),
                                     preferred_element_type=jnp.float32)
            a = ee / esum_b
            rsum = lax.dot_general(onef, a * sv, (((0,), (0,)), ((), ())),
                                   preferred_element_type=jnp.float32)
            q_star = jnp.concatenate([q, rsum], axis=1)
        o1 = jax.nn.relu(
            jnp.dot(q_star, w1[...], preferred_element_type=jnp.float32)
            + b1[...])
        o_ref[...] = (jnp.dot(o1, w2[...], preferred_element_type=jnp.float32)
                      + b2[...])

    return pl.pallas_call(
        body,
        out_shape=jax.ShapeDtypeStruct((_NG, 128), jnp.float32),
    )(s, batch2, wih2t, whh2t, bgr, w1t, b1r, w2t, b2r)


def kernel(x, edge_index, edge_attr, batch, W0, b0, Wn1, bn1, Wn2, bn2, root,
           conv_bias, Wih, Whh, bih, bhh, Wih2, Whh2, bih2, bhh2, W1, b1, W2,
           b2):
    src = jnp.concatenate(
        [edge_index[0], jnp.zeros((_EPAD - _E,), jnp.int32)]).reshape(
            _NW, _CPW, _CHUNK)
    dst = jnp.concatenate(
        [edge_index[1], jnp.full((_EPAD - _E,), _NPAD - 1, jnp.int32)]).reshape(
            _NW, _CPW, _CHUNK)
    eap = jnp.concatenate(
        [edge_attr, jnp.zeros((_EPAD - _E, _EDGE_F), jnp.float32)])
    jj = jnp.arange(_DIM * _DIM, dtype=jnp.int32)
    rep = (jj[None, :] // _DIM
           == jnp.arange(128, dtype=jnp.int32)[:, None]).astype(jnp.float32)
    summ = (jj[:, None] % _DIM
            == jnp.arange(_DIM, dtype=jnp.int32)[None, :]).astype(jnp.float32)
    zeros_tab = jnp.zeros((_NPAD, _DIM), jnp.float32)
    ones_vals = jnp.ones((_EPAD, _DIM), jnp.float32)
    w2t_pad = jnp.concatenate(
        [W2.T, jnp.zeros((_DIM, 128 - W2.shape[0]), jnp.float32)], axis=1)
    b2_pad = jnp.concatenate(
        [b2, jnp.zeros((128 - b2.shape[0],), jnp.float32)])[None, :]
    w0t_pad = jnp.concatenate(
        [W0.T, jnp.zeros((_NODE_F, 128 - _DIM), jnp.float32)], axis=1)
    b0_pad = jnp.concatenate(
        [b0, jnp.zeros((128 - _DIM,), jnp.float32)])[None, :]

    s = _mm_bias_relu(x, w0t_pad, b0_pad, 1000)
    hid = _mm_bias_relu(eap, Wn1.T, bn1[None, :], 2048)

    degp = _sc_scatter_add(ones_vals, dst, zeros_tab).reshape(2, _NPAD, _DIM)

    for _ in range(_ITERS):
        xj = _sc_gather(s, src)
        msg = _msg_kernel(hid, xj, Wn2.T, bn2[None, :], rep, summ)
        aggp = _sc_scatter_add(msg, dst, zeros_tab).reshape(2, _NPAD, _DIM)
        s = _node_update(aggp, degp, s, root, conv_bias[None, :],
                         Wih.T, Whh.T, bih[None, :], bhh[None, :])

    out = _set2set(s, batch[:, None], Wih2.T, Whh2.T,
                   (bih2 + bhh2)[None, :], W1.T, b1[None, :], w2t_pad, b2_pad)
    return out[:, :1]


# speedup vs baseline: 1.9738x; 1.0012x over previous
import functools

import jax
import jax.numpy as jnp
from jax import lax
from jax.experimental import pallas as pl
from jax.experimental.pallas import tpu as pltpu
from jax.experimental.pallas import tpu_sc as plsc

_N = 10000
_E = 160000
_NODE_F = 128
_EDGE_F = 16
_DIM = 32
_NN_HID = 128
_NG = 64
_ITERS = 3

_NW = 32
_CHUNK = 128
_EPAD = 163840
_CPW = _EPAD // _NW // _CHUNK
_NPAD = 10240
_ROWS_PT = _NPAD // 16
_AK = 2
_GK = 2


def _sc_gather(table, idx2):
    mesh = plsc.VectorSubcoreMesh(core_axis_name="c", subcore_axis_name="s")

    @functools.partial(
        pl.kernel, mesh=mesh,
        out_type=jax.ShapeDtypeStruct((_EPAD, 128), jnp.float32),
        scratch_types=[
            pltpu.VMEM((_CPW, _CHUNK), jnp.int32),
            pltpu.VMEM((_CHUNK, 128), jnp.float32),
            pltpu.VMEM((_CHUNK, 128), jnp.float32),
            pltpu.SemaphoreType.DMA,
            pltpu.SemaphoreType.DMA,
            pltpu.SemaphoreType.DMA,
        ],
    )
    def k(table_hbm, idx_hbm, out_hbm, idx_v, rows_v0, rows_v1,
          sem_g0, sem_g1, sem_w):
        wid = lax.axis_index("s") * 2 + lax.axis_index("c")
        pltpu.sync_copy(idx_hbm.at[wid], idx_v)
        base = wid * (_CPW * _CHUNK)

        def body(ci, carry):
            pltpu.async_copy(table_hbm.at[idx_v.at[ci]], rows_v0,
                             sem_g0).wait()
            pltpu.sync_copy(rows_v0,
                            out_hbm.at[pl.ds(base + ci * _CHUNK, _CHUNK)])
            return carry

        lax.fori_loop(0, _CPW, body, 0)

    return k(table, idx2)


def _sc_scatter_add(vals, idx2, zeros):
    mesh = plsc.VectorSubcoreMesh(core_axis_name="c", subcore_axis_name="s")

    @functools.partial(
        pl.kernel, mesh=mesh,
        out_type=jax.ShapeDtypeStruct((2 * _NPAD, _DIM), jnp.float32),
        scratch_types=[
            pltpu.VMEM((_CPW, _CHUNK), jnp.int32),
            pltpu.VMEM((_CHUNK, _DIM), jnp.float32),
            pltpu.VMEM((_CHUNK, _DIM), jnp.float32),
            pltpu.VMEM_SHARED((_NPAD, _DIM), jnp.float32),
            pltpu.SemaphoreType.DMA,
            pltpu.SemaphoreType.DMA,
            pltpu.SemaphoreType.DMA,
        ],
    )
    def k(vals_hbm, idx_hbm, zeros_hbm, out_hbm, idx_v, vals_v0, vals_v1,
          table, sem_r0, sem_r1, sem_a):
        c = lax.axis_index("c")
        s = lax.axis_index("s")
        wid = s * 2 + c
        r0 = s * _ROWS_PT
        pltpu.sync_copy(zeros_hbm.at[pl.ds(r0, _ROWS_PT)],
                        table.at[pl.ds(r0, _ROWS_PT)])
        pltpu.sync_copy(idx_hbm.at[wid], idx_v)
        plsc.subcore_barrier()
        base = wid * (_CPW * _CHUNK)
        sems = (sem_r0, sem_r1)
        bufs = (vals_v0, vals_v1)
        pend = pltpu.async_copy(vals_hbm.at[pl.ds(base, _CHUNK)], bufs[0],
                                sems[0])
        for g in range(_CPW):
            if g + 1 < _CPW:
                nxt = pltpu.async_copy(
                    vals_hbm.at[pl.ds(base + (g + 1) * _CHUNK, _CHUNK)],
                    bufs[(g + 1) % 2], sems[(g + 1) % 2])
            pend.wait()
            pltpu.sync_copy(bufs[g % 2], table.at[idx_v.at[g]], add=True)
            if g + 1 < _CPW:
                pend = nxt
        plsc.subcore_barrier()
        pltpu.sync_copy(table.at[pl.ds(r0, _ROWS_PT)],
                        out_hbm.at[pl.ds(c * _NPAD + r0, _ROWS_PT)])

    return k(vals, idx2, zeros)


def _mm_bias_relu(x, wt, br, blk):
    rows, kdim = x.shape
    cols = wt.shape[1]

    def body(x_ref, w_ref, b_ref, o_ref):
        o_ref[...] = jax.nn.relu(
            jnp.dot(x_ref[...], w_ref[...], preferred_element_type=jnp.float32)
            + b_ref[...])

    return pl.pallas_call(
        body,
        grid=(rows // blk,),
        in_specs=[
            pl.BlockSpec((blk, kdim), lambda i: (i, 0)),
            pl.BlockSpec((kdim, cols), lambda i: (0, 0)),
            pl.BlockSpec((1, cols), lambda i: (0, 0)),
        ],
        out_specs=pl.BlockSpec((blk, cols), lambda i: (i, 0)),
        out_shape=jax.ShapeDtypeStruct((rows, cols), jnp.float32),
    )(x, wt, br)


def _msg_kernel(hid, xj, wn2t, bn2r, rep, summ):
    blk = 640

    def body(h_ref, x_ref, w_ref, b_ref, r_ref, s_ref, o_ref):
        we = jnp.dot(h_ref[...], w_ref[...], preferred_element_type=jnp.float32)
        we = we + b_ref[...]
        xr = jnp.dot(x_ref[...], r_ref[...], preferred_element_type=jnp.float32)
        o_ref[...] = jnp.dot(we * xr, s_ref[...],
                             preferred_element_type=jnp.float32)

    return pl.pallas_call(
        body,
        grid=(_EPAD // blk,),
        in_specs=[
            pl.BlockSpec((blk, _NN_HID), lambda i: (i, 0)),
            pl.BlockSpec((blk, 128), lambda i: (i, 0)),
            pl.BlockSpec((_NN_HID, _DIM * _DIM), lambda i: (0, 0)),
            pl.BlockSpec((1, _DIM * _DIM), lambda i: (0, 0)),
            pl.BlockSpec((128, _DIM * _DIM), lambda i: (0, 0)),
            pl.BlockSpec((_DIM * _DIM, _DIM), lambda i: (0, 0)),
        ],
        out_specs=pl.BlockSpec((blk, _DIM), lambda i: (i, 0)),
        out_shape=jax.ShapeDtypeStruct((_EPAD, _DIM), jnp.float32),
    )(hid, xj, wn2t, bn2r, rep, summ)


def _node_update(aggp, degp, s, root, cbr, wiht, whht, bihr, bhhr):
    blk = 1000

    def body(a0, a1, d0, d1, s_ref, rt, cb, wi, wh, bi, bh, o_ref):
        deg = d0[0] + d1[0]
        denom = jnp.maximum(deg, 1.0)
        agg = (a0[0] + a1[0]) / denom
        sv = s_ref[...][:, :_DIM]
        m = jax.nn.relu(
            jnp.dot(sv, rt[...], preferred_element_type=jnp.float32) + agg
            + cb[...])
        gi = jnp.dot(m, wi[...], preferred_element_type=jnp.float32) + bi[...]
        gh = jnp.dot(sv, wh[...], preferred_element_type=jnp.float32) + bh[...]
        r = jax.nn.sigmoid(gi[:, :_DIM] + gh[:, :_DIM])
        z = jax.nn.sigmoid(gi[:, _DIM:2 * _DIM] + gh[:, _DIM:2 * _DIM])
        n = jnp.tanh(gi[:, 2 * _DIM:] + r * gh[:, 2 * _DIM:])
        h_new = (1.0 - z) * n + z * sv
        o_ref[...] = jnp.concatenate(
            [h_new, jnp.zeros((h_new.shape[0], 128 - _DIM), jnp.float32)],
            axis=1)

    part = pl.BlockSpec((1, blk, _DIM), lambda i: (0, i, 0))
    part1 = pl.BlockSpec((1, blk, _DIM), lambda i: (1, i, 0))
    return pl.pallas_call(
        body,
        grid=(_N // blk,),
        in_specs=[
            part, part1, part, part1,
            pl.BlockSpec((blk, 128), lambda i: (i, 0)),
            pl.BlockSpec((_DIM, _DIM), lambda i: (0, 0)),
            pl.BlockSpec((1, _DIM), lambda i: (0, 0)),
            pl.BlockSpec((_DIM, 3 * _DIM), lambda i: (0, 0)),
            pl.BlockSpec((_DIM, 3 * _DIM), lambda i: (0, 0)),
            pl.BlockSpec((1, 3 * _DIM), lambda i: (0, 0)),
            pl.BlockSpec((1, 3 * _DIM), lambda i: (0, 0)),
        ],
        out_specs=pl.BlockSpec((blk, 128), lambda i: (i, 0)),
        out_shape=jax.ShapeDtypeStruct((_N, 128), jnp.float32),
    )(aggp, aggp, degp, degp, s, root, cbr, wiht, whht, bihr, bhhr)


def _set2set(s, batch2, wih2t, whh2t, bgr, w1t, b1r, w2t, b2r):
    def body(s_ref, b_ref, wi, wh, bg, w1, b1, w2, b2, o_ref):
        sv = s_ref[...][:, :_DIM]
        bcol = b_ref[...]
        gidx = lax.broadcasted_iota(jnp.int32, (1, _NG), 1)
        oneb = (bcol == gidx)
        onef = oneb.astype(jnp.float32)
        q_star = jnp.zeros((_NG, 2 * _DIM), jnp.float32)
        hs = jnp.zeros((_NG, _DIM), jnp.float32)
        cs = jnp.zeros((_NG, _DIM), jnp.float32)
        for _ in range(_ITERS):
            gates = (jnp.dot(q_star, wi[...], preferred_element_type=jnp.float32)
                     + jnp.dot(hs, wh[...], preferred_element_type=jnp.float32)
                     + bg[...])
            i_g = jax.nn.sigmoid(gates[:, :_DIM])
            f_g = jax.nn.sigmoid(gates[:, _DIM:2 * _DIM])
            g_g = jnp.tanh(gates[:, 2 * _DIM:3 * _DIM])
            o_g = jax.nn.sigmoid(gates[:, 3 * _DIM:])
            cs = f_g * cs + i_g * g_g
            hs = o_g * jnp.tanh(cs)
            q = hs
            qb = jnp.dot(onef, q, preferred_element_type=jnp.float32)
            e = jnp.sum(sv * qb, axis=1, keepdims=True)
            masked = jnp.where(oneb, e, -1e30)
            emax = jnp.max(masked, axis=0, keepdims=True)
            emax_b = lax.dot_general(onef, emax, (((1,), (1,)), ((), ())),
                                     preferred_element_type=jnp.float32)
            ee = jnp.exp(e - emax_b)
            esum = lax.dot_general(ee, onef, (((0,), (0,)), ((), ())),
                                   preferred_element_type=jnp.float32)
            esum_b = lax.dot_general(onef, esum, (((1,), (1,)), ((), ())),
                                     preferred_element_type=jnp.float32)
            a = ee / esum_b
            rsum = lax.dot_general(onef, a * sv, (((0,), (0,)), ((), ())),
                                   preferred_element_type=jnp.float32)
            q_star = jnp.concatenate([q, rsum], axis=1)
        o1 = jax.nn.relu(
            jnp.dot(q_star, w1[...], preferred_element_type=jnp.float32)
            + b1[...])
        o_ref[...] = (jnp.dot(o1, w2[...], preferred_element_type=jnp.float32)
                      + b2[...])

    return pl.pallas_call(
        body,
        out_shape=jax.ShapeDtypeStruct((_NG, 128), jnp.float32),
    )(s, batch2, wih2t, whh2t, bgr, w1t, b1r, w2t, b2r)


def kernel(x, edge_index, edge_attr, batch, W0, b0, Wn1, bn1, Wn2, bn2, root,
           conv_bias, Wih, Whh, bih, bhh, Wih2, Whh2, bih2, bhh2, W1, b1, W2,
           b2):
    src = jnp.concatenate(
        [edge_index[0], jnp.zeros((_EPAD - _E,), jnp.int32)]).reshape(
            _NW, _CPW, _CHUNK)
    dst = jnp.concatenate(
        [edge_index[1], jnp.full((_EPAD - _E,), _NPAD - 1, jnp.int32)]).reshape(
            _NW, _CPW, _CHUNK)
    eap = jnp.concatenate(
        [edge_attr, jnp.zeros((_EPAD - _E, _EDGE_F), jnp.float32)])
    jj = jnp.arange(_DIM * _DIM, dtype=jnp.int32)
    rep = (jj[None, :] // _DIM
           == jnp.arange(128, dtype=jnp.int32)[:, None]).astype(jnp.float32)
    summ = (jj[:, None] % _DIM
            == jnp.arange(_DIM, dtype=jnp.int32)[None, :]).astype(jnp.float32)
    zeros_tab = jnp.zeros((_NPAD, _DIM), jnp.float32)
    ones_vals = jnp.ones((_EPAD, _DIM), jnp.float32)
    w2t_pad = jnp.concatenate(
        [W2.T, jnp.zeros((_DIM, 128 - W2.shape[0]), jnp.float32)], axis=1)
    b2_pad = jnp.concatenate(
        [b2, jnp.zeros((128 - b2.shape[0],), jnp.float32)])[None, :]
    w0t_pad = jnp.concatenate(
        [W0.T, jnp.zeros((_NODE_F, 128 - _DIM), jnp.float32)], axis=1)
    b0_pad = jnp.concatenate(
        [b0, jnp.zeros((128 - _DIM,), jnp.float32)])[None, :]

    s = _mm_bias_relu(x, w0t_pad, b0_pad, 1000)
    hid = _mm_bias_relu(eap, Wn1.T, bn1[None, :], 2048)

    degp = _sc_scatter_add(ones_vals, dst, zeros_tab).reshape(2, _NPAD, _DIM)

    for _ in range(_ITERS):
        xj = _sc_gather(s, src)
        msg = _msg_kernel(hid, xj, Wn2.T, bn2[None, :], rep, summ)
        aggp = _sc_scatter_add(msg, dst, zeros_tab).reshape(2, _NPAD, _DIM)
        s = _node_update(aggp, degp, s, root, conv_bias[None, :],
                         Wih.T, Whh.T, bih[None, :], bhh[None, :])

    out = _set2set(s, batch[:, None], Wih2.T, Whh2.T,
                   (bih2 + bhh2)[None, :], W1.T, b1[None, :], w2t_pad, b2_pad)
    return out[:, :1]
